# Initial kernel scaffold; baseline (speedup 1.0000x reference)
import jax
import jax.numpy as jnp
from jax.experimental import pallas as pl


def kernel(x, edge_index, W1, b1, W2, b2, W3, b3):
    raise NotImplementedError("write your pallas kernel here")


# speedup vs baseline: 7.8064x; 7.8064x over previous
import functools

import jax
import jax.numpy as jnp
from jax import lax
from jax.experimental import pallas as pl
from jax.experimental.pallas import tpu as pltpu
from jax.experimental.pallas import tpu_sc as plsc

NC = 2
NS = 16
NW = NC * NS
CHUNK = 128


def _sc_mesh():
    return plsc.VectorSubcoreMesh(
        core_axis_name="c", subcore_axis_name="s",
        num_cores=NC, num_subcores=NS)


def _sc_edge_aggregate(y, src2, dst2, zeros, k_per_w):
    npad, d = y.shape
    rows_per_sub = npad // NS
    out_t = jax.ShapeDtypeStruct((npad, d), jnp.float32)

    @functools.partial(
        pl.kernel,
        mesh=_sc_mesh(),
        out_type=(out_t, out_t),
        compiler_params=pltpu.CompilerParams(use_tc_tiling_on_sc=False),
        scratch_types=[
            pltpu.VMEM_SHARED((npad, d), jnp.float32),
            pltpu.VMEM((k_per_w, CHUNK), jnp.int32),
            pltpu.VMEM((k_per_w, CHUNK), jnp.int32),
            pltpu.VMEM((CHUNK, d), jnp.float32),
            pltpu.SemaphoreType.DMA,
        ],
    )
    def body(y_hbm, s_hbm, d_hbm, z_hbm, out0, out1, acc, sidx, didx, ybuf,
             sem):
        cid = lax.axis_index("c")
        sid = lax.axis_index("s")
        w = cid * NS + sid
        rows = pl.ds(sid * rows_per_sub, rows_per_sub)
        pltpu.sync_copy(z_hbm.at[rows], acc.at[rows])
        pltpu.sync_copy(s_hbm.at[pl.ds(w * k_per_w, k_per_w)], sidx)
        pltpu.sync_copy(d_hbm.at[pl.ds(w * k_per_w, k_per_w)], didx)
        plsc.subcore_barrier()

        def chunk(j, carry):
            pltpu.async_copy(y_hbm.at[sidx.at[j]], ybuf, sem).wait()
            pltpu.sync_copy(ybuf, acc.at[didx.at[j]], add=True)
            return carry

        lax.fori_loop(0, k_per_w, chunk, 0)
        plsc.subcore_barrier()

        @pl.when(cid == 0)
        def _():
            pltpu.sync_copy(acc.at[rows], out0.at[rows])

        @pl.when(cid == 1)
        def _():
            pltpu.sync_copy(acc.at[rows], out1.at[rows])

    return body(y, src2, dst2, zeros)


DEG_W = 8


def _tc_dis(d0, d1, bm=640):
    npad = d0.shape[0]

    def body(a_ref, b_ref, o_ref):
        o_ref[...] = lax.rsqrt(a_ref[...][:, :1] + b_ref[...][:, :1] + 1.0)

    return pl.pallas_call(
        body,
        grid=(npad // bm,),
        in_specs=[pl.BlockSpec((bm, DEG_W), lambda i: (i, 0)),
                  pl.BlockSpec((bm, DEG_W), lambda i: (i, 0))],
        out_specs=pl.BlockSpec((bm, 1), lambda i: (i, 0)),
        out_shape=jax.ShapeDtypeStruct((npad, 1), jnp.float32),
    )(d0, d1)


def _tc_mm_scale(h, w, dis, bm=1024):
    npad, din = h.shape
    dout = w.shape[1]

    def body(h_ref, w_ref, dis_ref, o_ref):
        acc = lax.dot_general(
            h_ref[...], w_ref[...], (((1,), (0,)), ((), ())),
            preferred_element_type=jnp.float32,
            precision=lax.Precision.HIGHEST)
        o_ref[...] = acc * dis_ref[...]

    return pl.pallas_call(
        body,
        grid=(npad // bm,),
        in_specs=[pl.BlockSpec((bm, din), lambda i: (i, 0)),
                  pl.BlockSpec((din, dout), lambda i: (0, 0)),
                  pl.BlockSpec((bm, 1), lambda i: (i, 0))],
        out_specs=pl.BlockSpec((bm, dout), lambda i: (i, 0)),
        out_shape=jax.ShapeDtypeStruct((npad, dout), jnp.float32),
    )(h, w, dis)


def _tc_fuse(p0, p1, y, dis, b, out_rows, bm):
    d = y.shape[1]

    def body(a_ref, b_ref, y_ref, dis_ref, bias_ref, o_ref):
        v = dis_ref[...] * (a_ref[...] + b_ref[...] + y_ref[...])
        o_ref[...] = jnp.maximum(v + bias_ref[...], 0.0)

    blk = lambda i: (i, 0)
    return pl.pallas_call(
        body,
        grid=(out_rows // bm,),
        in_specs=[pl.BlockSpec((bm, d), blk),
                  pl.BlockSpec((bm, d), blk),
                  pl.BlockSpec((bm, d), blk),
                  pl.BlockSpec((bm, 1), blk),
                  pl.BlockSpec((1, d), lambda i: (0, 0))],
        out_specs=pl.BlockSpec((bm, d), blk),
        out_shape=jax.ShapeDtypeStruct((out_rows, d), jnp.float32),
    )(p0, p1, y, dis, b)


def _tc_transpose(z, bm=640):
    npad, d = z.shape

    def body(z_ref, o_ref):
        o_ref[...] = z_ref[...].T

    return pl.pallas_call(
        body,
        grid=(npad // bm,),
        in_specs=[pl.BlockSpec((bm, d), lambda i: (i, 0))],
        out_specs=pl.BlockSpec((d, bm), lambda i: (0, i)),
        out_shape=jax.ShapeDtypeStruct((d, npad), jnp.float32),
    )(z)


def _tc_ahat(z, zt, n, bm=200):
    d = z.shape[1]

    def body(zr_ref, zt_ref, o_ref):
        o_ref[...] = lax.dot_general(
            zr_ref[...], zt_ref[...], (((1,), (0,)), ((), ())),
            preferred_element_type=jnp.float32,
            precision=lax.Precision.HIGHEST)

    return pl.pallas_call(
        body,
        grid=(n // bm,),
        in_specs=[pl.BlockSpec((bm, d), lambda i: (i, 0)),
                  pl.BlockSpec((d, n), lambda i: (0, 0))],
        out_specs=pl.BlockSpec((bm, n), lambda i: (i, 0)),
        out_shape=jax.ShapeDtypeStruct((n, n), jnp.float32),
    )(z, zt)


def kernel(x, edge_index, W1, b1, W2, b2, W3, b3):
    n, d_in = x.shape
    e = edge_index.shape[1]
    d_h = W1.shape[1]

    npad = ((n + NW * 16 - 1) // (NW * 16)) * (NW * 16)
    unit = NW * CHUNK * 8
    epad = ((e + unit - 1) // unit) * unit
    k_per_w = epad // (NW * CHUNK)

    src = edge_index[0].astype(jnp.int32)
    dst = edge_index[1].astype(jnp.int32)
    padi = jnp.full((epad - e,), n, jnp.int32)
    src2 = jnp.concatenate([src, padi]).reshape(-1, CHUNK)
    dst2 = jnp.concatenate([dst, padi]).reshape(-1, CHUNK)

    x_pad = jnp.pad(x, ((0, npad - n), (0, 0)))
    ones_t = jnp.ones((npad, DEG_W), jnp.float32)
    zeros_d = jnp.zeros((npad, DEG_W), jnp.float32)
    zeros_h = jnp.zeros((npad, d_h), jnp.float32)
    zeros_in = jnp.zeros((npad, d_in), jnp.float32)

    g0, g1 = _sc_edge_aggregate(ones_t, dst2, dst2, zeros_d, k_per_w)
    dis = _tc_dis(g0, g1)

    y1 = _tc_mm_scale(x_pad, W1, dis)
    p0, p1 = _sc_edge_aggregate(y1, src2, dst2, zeros_h, k_per_w)
    h1 = _tc_fuse(p0, p1, y1, dis, b1.reshape(1, -1), npad, 1024)

    y2 = _tc_mm_scale(h1, W2, dis)
    q0, q1 = _sc_edge_aggregate(y2, src2, dst2, zeros_h, k_per_w)
    z = _tc_fuse(q0, q1, y2, dis, b2.reshape(1, -1), npad, 1024)

    y3 = _tc_mm_scale(z, W3, dis)
    r0, r1 = _sc_edge_aggregate(y3, src2, dst2, zeros_in, k_per_w)
    x_hat = _tc_fuse(r0, r1, y3, dis, b3.reshape(1, -1), n, 400)

    zt = _tc_transpose(z)[:, :n]
    a_hat = _tc_ahat(z, zt, n)
    return (a_hat, x_hat)


# speedup vs baseline: 8.8985x; 1.1399x over previous
import functools

import jax
import jax.numpy as jnp
from jax import lax
from jax.experimental import pallas as pl
from jax.experimental.pallas import tpu as pltpu
from jax.experimental.pallas import tpu_sc as plsc

NC = 2
NS = 16
NW = NC * NS
CHUNK = 128


def _sc_mesh():
    return plsc.VectorSubcoreMesh(
        core_axis_name="c", subcore_axis_name="s",
        num_cores=NC, num_subcores=NS)


def _sc_edge_aggregate(y, src2, dst2, zeros, k_per_w, chunk):
    npad, d = y.shape
    rows_per_sub = npad // NS
    out_t = jax.ShapeDtypeStruct((npad, d), jnp.float32)

    @functools.partial(
        pl.kernel,
        mesh=_sc_mesh(),
        out_type=(out_t, out_t),
        compiler_params=pltpu.CompilerParams(use_tc_tiling_on_sc=False),
        scratch_types=[
            pltpu.VMEM_SHARED((npad, d), jnp.float32),
            pltpu.VMEM((k_per_w, chunk), jnp.int32),
            pltpu.VMEM((k_per_w, chunk), jnp.int32),
            pltpu.VMEM((chunk, d), jnp.float32),
            pltpu.VMEM((chunk, d), jnp.float32),
            pltpu.SemaphoreType.DMA,
            pltpu.SemaphoreType.DMA,
        ],
    )
    def body(y_hbm, s_hbm, d_hbm, z_hbm, out0, out1, acc, sidx, didx, ybuf0,
             ybuf1, sem0, sem1):
        cid = lax.axis_index("c")
        sid = lax.axis_index("s")
        w = cid * NS + sid
        rows = pl.ds(sid * rows_per_sub, rows_per_sub)
        pltpu.sync_copy(z_hbm.at[rows], acc.at[rows])
        pltpu.sync_copy(s_hbm.at[pl.ds(w * k_per_w, k_per_w)], sidx)
        pltpu.sync_copy(d_hbm.at[pl.ds(w * k_per_w, k_per_w)], didx)
        plsc.subcore_barrier()

        bufs = (ybuf0, ybuf1)
        sems = (sem0, sem1)
        nb = 2
        for j in range(min(nb, k_per_w)):
            pltpu.async_copy(y_hbm.at[sidx.at[j]], bufs[j % nb], sems[j % nb])
        for j in range(k_per_w):
            b = j % nb
            pltpu.make_async_copy(y_hbm.at[sidx.at[j]], bufs[b],
                                  sems[b]).wait()
            pltpu.sync_copy(bufs[b], acc.at[didx.at[j]], add=True)
            if j + nb < k_per_w:
                pltpu.async_copy(y_hbm.at[sidx.at[j + nb]], bufs[b], sems[b])
        plsc.subcore_barrier()

        @pl.when(cid == 0)
        def _():
            pltpu.sync_copy(acc.at[rows], out0.at[rows])

        @pl.when(cid == 1)
        def _():
            pltpu.sync_copy(acc.at[rows], out1.at[rows])

    return body(y, src2, dst2, zeros)


DEG_W = 8


def _tc_dis(d0, d1, bm=640):
    npad = d0.shape[0]

    def body(a_ref, b_ref, o_ref):
        o_ref[...] = lax.rsqrt(a_ref[...][:, :1] + b_ref[...][:, :1] + 1.0)

    return pl.pallas_call(
        body,
        grid=(npad // bm,),
        in_specs=[pl.BlockSpec((bm, DEG_W), lambda i: (i, 0)),
                  pl.BlockSpec((bm, DEG_W), lambda i: (i, 0))],
        out_specs=pl.BlockSpec((bm, 1), lambda i: (i, 0)),
        out_shape=jax.ShapeDtypeStruct((npad, 1), jnp.float32),
    )(d0, d1)


def _tc_mm_scale(h, w, dis, bm=1024):
    npad, din = h.shape
    dout = w.shape[1]

    def body(h_ref, w_ref, dis_ref, o_ref):
        acc = lax.dot_general(
            h_ref[...], w_ref[...], (((1,), (0,)), ((), ())),
            preferred_element_type=jnp.float32,
            precision=lax.Precision.HIGHEST)
        o_ref[...] = acc * dis_ref[...]

    return pl.pallas_call(
        body,
        grid=(npad // bm,),
        in_specs=[pl.BlockSpec((bm, din), lambda i: (i, 0)),
                  pl.BlockSpec((din, dout), lambda i: (0, 0)),
                  pl.BlockSpec((bm, 1), lambda i: (i, 0))],
        out_specs=pl.BlockSpec((bm, dout), lambda i: (i, 0)),
        out_shape=jax.ShapeDtypeStruct((npad, dout), jnp.float32),
    )(h, w, dis)


def _tc_fuse(p0, p1, y, dis, b, out_rows, bm):
    d = y.shape[1]

    def body(a_ref, b_ref, y_ref, dis_ref, bias_ref, o_ref):
        v = dis_ref[...] * (a_ref[...] + b_ref[...] + y_ref[...])
        o_ref[...] = jnp.maximum(v + bias_ref[...], 0.0)

    blk = lambda i: (i, 0)
    return pl.pallas_call(
        body,
        grid=(out_rows // bm,),
        in_specs=[pl.BlockSpec((bm, d), blk),
                  pl.BlockSpec((bm, d), blk),
                  pl.BlockSpec((bm, d), blk),
                  pl.BlockSpec((bm, 1), blk),
                  pl.BlockSpec((1, d), lambda i: (0, 0))],
        out_specs=pl.BlockSpec((bm, d), blk),
        out_shape=jax.ShapeDtypeStruct((out_rows, d), jnp.float32),
    )(p0, p1, y, dis, b)


def _tc_transpose(z, bm=640):
    npad, d = z.shape

    def body(z_ref, o_ref):
        o_ref[...] = z_ref[...].T

    return pl.pallas_call(
        body,
        grid=(npad // bm,),
        in_specs=[pl.BlockSpec((bm, d), lambda i: (i, 0))],
        out_specs=pl.BlockSpec((d, bm), lambda i: (0, i)),
        out_shape=jax.ShapeDtypeStruct((d, npad), jnp.float32),
    )(z)


def _tc_ahat(z, zt, n, bm=200):
    d = z.shape[1]

    def body(zr_ref, zt_ref, o_ref):
        o_ref[...] = lax.dot_general(
            zr_ref[...], zt_ref[...], (((1,), (0,)), ((), ())),
            preferred_element_type=jnp.float32,
            precision=lax.Precision.HIGHEST)

    return pl.pallas_call(
        body,
        grid=(n // bm,),
        in_specs=[pl.BlockSpec((bm, d), lambda i: (i, 0)),
                  pl.BlockSpec((d, n), lambda i: (0, 0))],
        out_specs=pl.BlockSpec((bm, n), lambda i: (i, 0)),
        out_shape=jax.ShapeDtypeStruct((n, n), jnp.float32),
    )(z, zt)


def kernel(x, edge_index, W1, b1, W2, b2, W3, b3):
    n, d_in = x.shape
    e = edge_index.shape[1]
    d_h = W1.shape[1]

    npad = ((n + NW * 16 - 1) // (NW * 16)) * (NW * 16)
    unit = NW * CHUNK * 8
    epad = ((e + unit - 1) // unit) * unit
    k_per_w = epad // (NW * CHUNK)

    src = edge_index[0].astype(jnp.int32)
    dst = edge_index[1].astype(jnp.int32)
    padi = jnp.full((epad - e,), n, jnp.int32)
    src_flat = jnp.concatenate([src, padi])
    dst_flat = jnp.concatenate([dst, padi])
    src2 = src_flat.reshape(-1, CHUNK)
    dst2 = dst_flat.reshape(-1, CHUNK)
    CH3 = 64
    src3 = src_flat.reshape(-1, CH3)
    dst3 = dst_flat.reshape(-1, CH3)
    k3_per_w = epad // (NW * CH3)

    x_pad = jnp.pad(x, ((0, npad - n), (0, 0)))
    ones_t = jnp.ones((npad, DEG_W), jnp.float32)
    zeros_d = jnp.zeros((npad, DEG_W), jnp.float32)
    zeros_h = jnp.zeros((npad, d_h), jnp.float32)
    zeros_in = jnp.zeros((npad, d_in), jnp.float32)

    g0, g1 = _sc_edge_aggregate(ones_t, dst2, dst2, zeros_d, k_per_w, CHUNK)
    dis = _tc_dis(g0, g1)

    y1 = _tc_mm_scale(x_pad, W1, dis)
    p0, p1 = _sc_edge_aggregate(y1, src2, dst2, zeros_h, k_per_w, CHUNK)
    h1 = _tc_fuse(p0, p1, y1, dis, b1.reshape(1, -1), npad, 1024)

    y2 = _tc_mm_scale(h1, W2, dis)
    q0, q1 = _sc_edge_aggregate(y2, src2, dst2, zeros_h, k_per_w, CHUNK)
    z = _tc_fuse(q0, q1, y2, dis, b2.reshape(1, -1), npad, 1024)

    y3 = _tc_mm_scale(z, W3, dis)
    r0, r1 = _sc_edge_aggregate(y3, src3, dst3, zeros_in, k3_per_w, CH3)
    x_hat = _tc_fuse(r0, r1, y3, dis, b3.reshape(1, -1), n, 400)

    zt = _tc_transpose(z)[:, :n]
    a_hat = _tc_ahat(z, zt, n)
    return (a_hat, x_hat)


# speedup vs baseline: 9.0908x; 1.0216x over previous
import functools

import jax
import jax.numpy as jnp
from jax import lax
from jax.experimental import pallas as pl
from jax.experimental.pallas import tpu as pltpu
from jax.experimental.pallas import tpu_sc as plsc

NC = 2
NS = 16
NW = NC * NS
CHUNK = 128


def _sc_mesh():
    return plsc.VectorSubcoreMesh(
        core_axis_name="c", subcore_axis_name="s",
        num_cores=NC, num_subcores=NS)


def _sc_edge_aggregate(y, src2, dst2, zeros, k_per_w, chunk, nb):
    npad, d = y.shape
    rows_per_sub = npad // NS
    out_t = jax.ShapeDtypeStruct((npad, d), jnp.float32)

    @functools.partial(
        pl.kernel,
        mesh=_sc_mesh(),
        out_type=(out_t, out_t),
        compiler_params=pltpu.CompilerParams(use_tc_tiling_on_sc=False),
        scratch_types=(
            [
                pltpu.VMEM_SHARED((npad, d), jnp.float32),
                pltpu.VMEM((k_per_w, chunk), jnp.int32),
                pltpu.VMEM((k_per_w, chunk), jnp.int32),
            ]
            + [pltpu.VMEM((chunk, d), jnp.float32)] * nb
            + [pltpu.SemaphoreType.DMA] * nb
        ),
    )
    def body(y_hbm, s_hbm, d_hbm, z_hbm, out0, out1, acc, sidx, didx, *ring):
        bufs = ring[:nb]
        sems = ring[nb:]
        cid = lax.axis_index("c")
        sid = lax.axis_index("s")
        w = cid * NS + sid
        rows = pl.ds(sid * rows_per_sub, rows_per_sub)
        pltpu.sync_copy(z_hbm.at[rows], acc.at[rows])
        pltpu.sync_copy(s_hbm.at[pl.ds(w * k_per_w, k_per_w)], sidx)
        pltpu.sync_copy(d_hbm.at[pl.ds(w * k_per_w, k_per_w)], didx)
        plsc.subcore_barrier()

        for j in range(min(nb, k_per_w)):
            pltpu.async_copy(y_hbm.at[sidx.at[j]], bufs[j % nb], sems[j % nb])
        for j in range(k_per_w):
            b = j % nb
            pltpu.make_async_copy(y_hbm.at[sidx.at[j]], bufs[b],
                                  sems[b]).wait()
            pltpu.sync_copy(bufs[b], acc.at[didx.at[j]], add=True)
            if j + nb < k_per_w:
                pltpu.async_copy(y_hbm.at[sidx.at[j + nb]], bufs[b], sems[b])
        plsc.subcore_barrier()

        @pl.when(cid == 0)
        def _():
            pltpu.sync_copy(acc.at[rows], out0.at[rows])

        @pl.when(cid == 1)
        def _():
            pltpu.sync_copy(acc.at[rows], out1.at[rows])

    return body(y, src2, dst2, zeros)


DEG_W = 8


def _tc_dis(d0, d1, bm=640):
    npad = d0.shape[0]

    def body(a_ref, b_ref, o_ref):
        o_ref[...] = lax.rsqrt(a_ref[...][:, :1] + b_ref[...][:, :1] + 1.0)

    return pl.pallas_call(
        body,
        grid=(npad // bm,),
        in_specs=[pl.BlockSpec((bm, DEG_W), lambda i: (i, 0)),
                  pl.BlockSpec((bm, DEG_W), lambda i: (i, 0))],
        out_specs=pl.BlockSpec((bm, 1), lambda i: (i, 0)),
        out_shape=jax.ShapeDtypeStruct((npad, 1), jnp.float32),
    )(d0, d1)


def _tc_mm_scale(h, w, dis, bm=1024):
    npad, din = h.shape
    dout = w.shape[1]

    def body(h_ref, w_ref, dis_ref, o_ref):
        acc = lax.dot_general(
            h_ref[...], w_ref[...], (((1,), (0,)), ((), ())),
            preferred_element_type=jnp.float32,
            precision=lax.Precision.HIGHEST)
        o_ref[...] = acc * dis_ref[...]

    return pl.pallas_call(
        body,
        grid=(npad // bm,),
        in_specs=[pl.BlockSpec((bm, din), lambda i: (i, 0)),
                  pl.BlockSpec((din, dout), lambda i: (0, 0)),
                  pl.BlockSpec((bm, 1), lambda i: (i, 0))],
        out_specs=pl.BlockSpec((bm, dout), lambda i: (i, 0)),
        out_shape=jax.ShapeDtypeStruct((npad, dout), jnp.float32),
    )(h, w, dis)


def _tc_fuse(p0, p1, y, dis, b, out_rows, bm):
    d = y.shape[1]

    def body(a_ref, b_ref, y_ref, dis_ref, bias_ref, o_ref):
        v = dis_ref[...] * (a_ref[...] + b_ref[...] + y_ref[...])
        o_ref[...] = jnp.maximum(v + bias_ref[...], 0.0)

    blk = lambda i: (i, 0)
    return pl.pallas_call(
        body,
        grid=(out_rows // bm,),
        in_specs=[pl.BlockSpec((bm, d), blk),
                  pl.BlockSpec((bm, d), blk),
                  pl.BlockSpec((bm, d), blk),
                  pl.BlockSpec((bm, 1), blk),
                  pl.BlockSpec((1, d), lambda i: (0, 0))],
        out_specs=pl.BlockSpec((bm, d), blk),
        out_shape=jax.ShapeDtypeStruct((out_rows, d), jnp.float32),
    )(p0, p1, y, dis, b)


def _tc_transpose(z, bm=640):
    npad, d = z.shape

    def body(z_ref, o_ref):
        o_ref[...] = z_ref[...].T

    return pl.pallas_call(
        body,
        grid=(npad // bm,),
        in_specs=[pl.BlockSpec((bm, d), lambda i: (i, 0))],
        out_specs=pl.BlockSpec((d, bm), lambda i: (0, i)),
        out_shape=jax.ShapeDtypeStruct((d, npad), jnp.float32),
    )(z)


def _tc_ahat(z, zt, n, bm=200):
    d = z.shape[1]

    def body(zr_ref, zt_ref, o_ref):
        o_ref[...] = lax.dot_general(
            zr_ref[...], zt_ref[...], (((1,), (0,)), ((), ())),
            preferred_element_type=jnp.float32,
            precision=lax.Precision.HIGHEST)

    return pl.pallas_call(
        body,
        grid=(n // bm,),
        in_specs=[pl.BlockSpec((bm, d), lambda i: (i, 0)),
                  pl.BlockSpec((d, n), lambda i: (0, 0))],
        out_specs=pl.BlockSpec((bm, n), lambda i: (i, 0)),
        out_shape=jax.ShapeDtypeStruct((n, n), jnp.float32),
    )(z, zt)


def kernel(x, edge_index, W1, b1, W2, b2, W3, b3):
    n, d_in = x.shape
    e = edge_index.shape[1]
    d_h = W1.shape[1]

    npad = ((n + NW * 16 - 1) // (NW * 16)) * (NW * 16)
    unit = NW * CHUNK * 8
    epad = ((e + unit - 1) // unit) * unit
    k_per_w = epad // (NW * CHUNK)

    src = edge_index[0].astype(jnp.int32)
    dst = edge_index[1].astype(jnp.int32)
    padi = jnp.full((epad - e,), n, jnp.int32)
    src_flat = jnp.concatenate([src, padi])
    dst_flat = jnp.concatenate([dst, padi])
    src2 = src_flat.reshape(-1, CHUNK)
    dst2 = dst_flat.reshape(-1, CHUNK)
    CH3 = 64
    src3 = src_flat.reshape(-1, CH3)
    dst3 = dst_flat.reshape(-1, CH3)
    k3_per_w = epad // (NW * CH3)

    x_pad = jnp.pad(x, ((0, npad - n), (0, 0)))
    ones_t = jnp.ones((npad, DEG_W), jnp.float32)
    zeros_d = jnp.zeros((npad, DEG_W), jnp.float32)
    zeros_h = jnp.zeros((npad, d_h), jnp.float32)
    zeros_in = jnp.zeros((npad, d_in), jnp.float32)

    g0, g1 = _sc_edge_aggregate(ones_t, dst2, dst2, zeros_d, k_per_w, CHUNK, 4)
    dis = _tc_dis(g0, g1)

    y1 = _tc_mm_scale(x_pad, W1, dis)
    p0, p1 = _sc_edge_aggregate(y1, src2, dst2, zeros_h, k_per_w, CHUNK, 4)
    h1 = _tc_fuse(p0, p1, y1, dis, b1.reshape(1, -1), npad, 1024)

    y2 = _tc_mm_scale(h1, W2, dis)
    q0, q1 = _sc_edge_aggregate(y2, src2, dst2, zeros_h, k_per_w, CHUNK, 4)
    z = _tc_fuse(q0, q1, y2, dis, b2.reshape(1, -1), npad, 1024)

    y3 = _tc_mm_scale(z, W3, dis)
    r0, r1 = _sc_edge_aggregate(y3, src3, dst3, zeros_in, k3_per_w, CH3, 3)
    x_hat = _tc_fuse(r0, r1, y3, dis, b3.reshape(1, -1), n, 400)

    zt = _tc_transpose(z)[:, :n]
    a_hat = _tc_ahat(z, zt, n)
    return (a_hat, x_hat)


# speedup vs baseline: 9.1473x; 1.0062x over previous
import functools

import jax
import jax.numpy as jnp
from jax import lax
from jax.experimental import pallas as pl
from jax.experimental.pallas import tpu as pltpu
from jax.experimental.pallas import tpu_sc as plsc

NC = 2
NS = 16
NW = NC * NS
CHUNK = 128


def _sc_mesh():
    return plsc.VectorSubcoreMesh(
        core_axis_name="c", subcore_axis_name="s",
        num_cores=NC, num_subcores=NS)


def _sc_edge_aggregate(y, src2, dst2, zeros, k_per_w, chunk, nb):
    npad, d = y.shape
    rows_per_sub = npad // NS
    out_t = jax.ShapeDtypeStruct((npad, d), jnp.float32)

    @functools.partial(
        pl.kernel,
        mesh=_sc_mesh(),
        out_type=(out_t, out_t),
        compiler_params=pltpu.CompilerParams(use_tc_tiling_on_sc=False),
        scratch_types=(
            [
                pltpu.VMEM_SHARED((npad, d), jnp.float32),
                pltpu.VMEM((k_per_w, chunk), jnp.int32),
                pltpu.VMEM((k_per_w, chunk), jnp.int32),
            ]
            + [pltpu.VMEM((chunk, d), jnp.float32)] * nb
            + [pltpu.SemaphoreType.DMA] * nb
        ),
    )
    def body(y_hbm, s_hbm, d_hbm, z_hbm, out0, out1, acc, sidx, didx, *ring):
        bufs = ring[:nb]
        sems = ring[nb:]
        cid = lax.axis_index("c")
        sid = lax.axis_index("s")
        w = cid * NS + sid
        rows = pl.ds(sid * rows_per_sub, rows_per_sub)
        pltpu.sync_copy(z_hbm.at[rows], acc.at[rows])
        pltpu.sync_copy(s_hbm.at[pl.ds(w * k_per_w, k_per_w)], sidx)
        pltpu.sync_copy(d_hbm.at[pl.ds(w * k_per_w, k_per_w)], didx)
        plsc.subcore_barrier()

        for j in range(min(nb, k_per_w)):
            pltpu.async_copy(y_hbm.at[sidx.at[j]], bufs[j % nb], sems[j % nb])
        for j in range(k_per_w):
            b = j % nb
            pltpu.make_async_copy(y_hbm.at[sidx.at[j]], bufs[b],
                                  sems[b]).wait()
            pltpu.sync_copy(bufs[b], acc.at[didx.at[j]], add=True)
            if j + nb < k_per_w:
                pltpu.async_copy(y_hbm.at[sidx.at[j + nb]], bufs[b], sems[b])
        plsc.subcore_barrier()

        @pl.when(cid == 0)
        def _():
            pltpu.sync_copy(acc.at[rows], out0.at[rows])

        @pl.when(cid == 1)
        def _():
            pltpu.sync_copy(acc.at[rows], out1.at[rows])

    return body(y, src2, dst2, zeros)


DEG_W = 8


def _tc_dis(d0, d1, bm=640):
    npad = d0.shape[0]

    def body(a_ref, b_ref, o_ref):
        o_ref[...] = lax.rsqrt(a_ref[...][:, :1] + b_ref[...][:, :1] + 1.0)

    return pl.pallas_call(
        body,
        grid=(npad // bm,),
        in_specs=[pl.BlockSpec((bm, DEG_W), lambda i: (i, 0)),
                  pl.BlockSpec((bm, DEG_W), lambda i: (i, 0))],
        out_specs=pl.BlockSpec((bm, 1), lambda i: (i, 0)),
        out_shape=jax.ShapeDtypeStruct((npad, 1), jnp.float32),
    )(d0, d1)


def _tc_mm_scale(h, w, dis, bm=1024):
    npad, din = h.shape
    dout = w.shape[1]

    def body(h_ref, w_ref, dis_ref, o_ref):
        acc = lax.dot_general(
            h_ref[...], w_ref[...], (((1,), (0,)), ((), ())),
            preferred_element_type=jnp.float32,
            precision=lax.Precision.HIGHEST)
        o_ref[...] = acc * dis_ref[...]

    return pl.pallas_call(
        body,
        grid=(npad // bm,),
        in_specs=[pl.BlockSpec((bm, din), lambda i: (i, 0)),
                  pl.BlockSpec((din, dout), lambda i: (0, 0)),
                  pl.BlockSpec((bm, 1), lambda i: (i, 0))],
        out_specs=pl.BlockSpec((bm, dout), lambda i: (i, 0)),
        out_shape=jax.ShapeDtypeStruct((npad, dout), jnp.float32),
    )(h, w, dis)


def _tc_fuse(p0, p1, y, dis, b, out_rows, bm):
    d = y.shape[1]

    def body(a_ref, b_ref, y_ref, dis_ref, bias_ref, o_ref):
        v = dis_ref[...] * (a_ref[...] + b_ref[...] + y_ref[...])
        o_ref[...] = jnp.maximum(v + bias_ref[...], 0.0)

    blk = lambda i: (i, 0)
    return pl.pallas_call(
        body,
        grid=(out_rows // bm,),
        in_specs=[pl.BlockSpec((bm, d), blk),
                  pl.BlockSpec((bm, d), blk),
                  pl.BlockSpec((bm, d), blk),
                  pl.BlockSpec((bm, 1), blk),
                  pl.BlockSpec((1, d), lambda i: (0, 0))],
        out_specs=pl.BlockSpec((bm, d), blk),
        out_shape=jax.ShapeDtypeStruct((out_rows, d), jnp.float32),
    )(p0, p1, y, dis, b)


def _tc_split_t(z, bm=640):
    npad, d = z.shape

    def body(z_ref, zh_ref, zl_ref, zht_ref, zlt_ref):
        v = z_ref[...]
        h = v.astype(jnp.bfloat16)
        l = (v - h.astype(jnp.float32)).astype(jnp.bfloat16)
        zh_ref[...] = h
        zl_ref[...] = l
        zht_ref[...] = h.T
        zlt_ref[...] = l.T

    return pl.pallas_call(
        body,
        grid=(npad // bm,),
        in_specs=[pl.BlockSpec((bm, d), lambda i: (i, 0))],
        out_specs=[pl.BlockSpec((bm, d), lambda i: (i, 0)),
                   pl.BlockSpec((bm, d), lambda i: (i, 0)),
                   pl.BlockSpec((d, bm), lambda i: (0, i)),
                   pl.BlockSpec((d, bm), lambda i: (0, i))],
        out_shape=[jax.ShapeDtypeStruct((npad, d), jnp.bfloat16),
                   jax.ShapeDtypeStruct((npad, d), jnp.bfloat16),
                   jax.ShapeDtypeStruct((d, npad), jnp.bfloat16),
                   jax.ShapeDtypeStruct((d, npad), jnp.bfloat16)],
    )(z)


def _tc_ahat(zh, zl, zht, zlt, n, bm=200):
    d = zh.shape[1]

    def body(zh_ref, zl_ref, zht_ref, zlt_ref, o_ref):
        dims = (((1,), (0,)), ((), ()))
        hh = lax.dot_general(zh_ref[...], zht_ref[...], dims,
                             preferred_element_type=jnp.float32)
        hl = lax.dot_general(zh_ref[...], zlt_ref[...], dims,
                             preferred_element_type=jnp.float32)
        lh = lax.dot_general(zl_ref[...], zht_ref[...], dims,
                             preferred_element_type=jnp.float32)
        o_ref[...] = hh + hl + lh

    return pl.pallas_call(
        body,
        grid=(n // bm,),
        in_specs=[pl.BlockSpec((bm, d), lambda i: (i, 0)),
                  pl.BlockSpec((bm, d), lambda i: (i, 0)),
                  pl.BlockSpec((d, n), lambda i: (0, 0)),
                  pl.BlockSpec((d, n), lambda i: (0, 0))],
        out_specs=pl.BlockSpec((bm, n), lambda i: (i, 0)),
        out_shape=jax.ShapeDtypeStruct((n, n), jnp.float32),
    )(zh, zl, zht, zlt)


def kernel(x, edge_index, W1, b1, W2, b2, W3, b3):
    n, d_in = x.shape
    e = edge_index.shape[1]
    d_h = W1.shape[1]

    npad = ((n + NW * 16 - 1) // (NW * 16)) * (NW * 16)
    unit = NW * CHUNK * 8
    epad = ((e + unit - 1) // unit) * unit
    k_per_w = epad // (NW * CHUNK)

    src = edge_index[0].astype(jnp.int32)
    dst = edge_index[1].astype(jnp.int32)
    padi = jnp.full((epad - e,), n, jnp.int32)
    src_flat = jnp.concatenate([src, padi])
    dst_flat = jnp.concatenate([dst, padi])
    src2 = src_flat.reshape(-1, CHUNK)
    dst2 = dst_flat.reshape(-1, CHUNK)
    CH3 = 64
    src3 = src_flat.reshape(-1, CH3)
    dst3 = dst_flat.reshape(-1, CH3)
    k3_per_w = epad // (NW * CH3)

    x_pad = jnp.pad(x, ((0, npad - n), (0, 0)))
    ones_t = jnp.ones((npad, DEG_W), jnp.float32)
    zeros_d = jnp.zeros((npad, DEG_W), jnp.float32)
    zeros_h = jnp.zeros((npad, d_h), jnp.float32)
    zeros_in = jnp.zeros((npad, d_in), jnp.float32)

    g0, g1 = _sc_edge_aggregate(ones_t, dst2, dst2, zeros_d, k_per_w, CHUNK, 4)
    dis = _tc_dis(g0, g1)

    y1 = _tc_mm_scale(x_pad, W1, dis)
    p0, p1 = _sc_edge_aggregate(y1, src2, dst2, zeros_h, k_per_w, CHUNK, 4)
    h1 = _tc_fuse(p0, p1, y1, dis, b1.reshape(1, -1), npad, 1024)

    y2 = _tc_mm_scale(h1, W2, dis)
    q0, q1 = _sc_edge_aggregate(y2, src2, dst2, zeros_h, k_per_w, CHUNK, 4)
    z = _tc_fuse(q0, q1, y2, dis, b2.reshape(1, -1), npad, 1024)

    y3 = _tc_mm_scale(z, W3, dis)
    r0, r1 = _sc_edge_aggregate(y3, src3, dst3, zeros_in, k3_per_w, CH3, 3)
    x_hat = _tc_fuse(r0, r1, y3, dis, b3.reshape(1, -1), n, 400)

    zh, zl, zht, zlt = _tc_split_t(z)
    a_hat = _tc_ahat(zh, zl, zht[:, :n], zlt[:, :n], n)
    return (a_hat, x_hat)


# speedup vs baseline: 9.3232x; 1.0192x over previous
import functools

import jax
import jax.numpy as jnp
from jax import lax
from jax.experimental import pallas as pl
from jax.experimental.pallas import tpu as pltpu
from jax.experimental.pallas import tpu_sc as plsc

NC = 2
NS = 16
NW = NC * NS
CHUNK = 128


def _sc_mesh():
    return plsc.VectorSubcoreMesh(
        core_axis_name="c", subcore_axis_name="s",
        num_cores=NC, num_subcores=NS)


def _sc_edge_aggregate(y, src2, dst2, zeros, k_per_w, chunk, nb):
    npad, d = y.shape
    rows_per_sub = npad // NS
    out_t = jax.ShapeDtypeStruct((npad, d), jnp.float32)

    @functools.partial(
        pl.kernel,
        mesh=_sc_mesh(),
        out_type=(out_t, out_t),
        compiler_params=pltpu.CompilerParams(use_tc_tiling_on_sc=False),
        scratch_types=(
            [
                pltpu.VMEM_SHARED((npad, d), jnp.float32),
                pltpu.VMEM((k_per_w, chunk), jnp.int32),
                pltpu.VMEM((k_per_w, chunk), jnp.int32),
            ]
            + [pltpu.VMEM((chunk, d), jnp.float32)] * nb
            + [pltpu.SemaphoreType.DMA] * nb
        ),
    )
    def body(y_hbm, s_hbm, d_hbm, z_hbm, out0, out1, acc, sidx, didx, *ring):
        bufs = ring[:nb]
        sems = ring[nb:]
        cid = lax.axis_index("c")
        sid = lax.axis_index("s")
        w = cid * NS + sid
        rows = pl.ds(sid * rows_per_sub, rows_per_sub)
        pltpu.sync_copy(z_hbm.at[rows], acc.at[rows])
        pltpu.sync_copy(s_hbm.at[pl.ds(w * k_per_w, k_per_w)], sidx)
        pltpu.sync_copy(d_hbm.at[pl.ds(w * k_per_w, k_per_w)], didx)
        plsc.subcore_barrier()

        for j in range(min(nb, k_per_w)):
            pltpu.async_copy(y_hbm.at[sidx.at[j]], bufs[j % nb], sems[j % nb])
        for j in range(k_per_w):
            b = j % nb
            pltpu.make_async_copy(y_hbm.at[sidx.at[j]], bufs[b],
                                  sems[b]).wait()
            pltpu.sync_copy(bufs[b], acc.at[didx.at[j]], add=True)
            if j + nb < k_per_w:
                pltpu.async_copy(y_hbm.at[sidx.at[j + nb]], bufs[b], sems[b])
        plsc.subcore_barrier()

        @pl.when(cid == 0)
        def _():
            pltpu.sync_copy(acc.at[rows], out0.at[rows])

        @pl.when(cid == 1)
        def _():
            pltpu.sync_copy(acc.at[rows], out1.at[rows])

    return body(y, src2, dst2, zeros)


DEG_W = 8


def _tc_dis(d0, d1, bm=640):
    npad = d0.shape[0]

    def body(a_ref, b_ref, o_ref):
        o_ref[...] = lax.rsqrt(a_ref[...][:, :1] + b_ref[...][:, :1] + 1.0)

    return pl.pallas_call(
        body,
        grid=(npad // bm,),
        in_specs=[pl.BlockSpec((bm, DEG_W), lambda i: (i, 0)),
                  pl.BlockSpec((bm, DEG_W), lambda i: (i, 0))],
        out_specs=pl.BlockSpec((bm, 1), lambda i: (i, 0)),
        out_shape=jax.ShapeDtypeStruct((npad, 1), jnp.float32),
    )(d0, d1)


def _tc_mm_scale(h, w, dis, bm=1024):
    npad, din = h.shape
    dout = w.shape[1]

    def body(h_ref, w_ref, dis_ref, o_ref):
        acc = lax.dot_general(
            h_ref[...], w_ref[...], (((1,), (0,)), ((), ())),
            preferred_element_type=jnp.float32,
            precision=lax.Precision.HIGHEST)
        o_ref[...] = acc * dis_ref[...]

    return pl.pallas_call(
        body,
        grid=(npad // bm,),
        in_specs=[pl.BlockSpec((bm, din), lambda i: (i, 0)),
                  pl.BlockSpec((din, dout), lambda i: (0, 0)),
                  pl.BlockSpec((bm, 1), lambda i: (i, 0))],
        out_specs=pl.BlockSpec((bm, dout), lambda i: (i, 0)),
        out_shape=jax.ShapeDtypeStruct((npad, dout), jnp.float32),
    )(h, w, dis)


def _tc_fuse(p0, p1, y, dis, b, out_rows, bm):
    d = y.shape[1]

    def body(a_ref, b_ref, y_ref, dis_ref, bias_ref, o_ref):
        v = dis_ref[...] * (a_ref[...] + b_ref[...] + y_ref[...])
        o_ref[...] = jnp.maximum(v + bias_ref[...], 0.0)

    blk = lambda i: (i, 0)
    return pl.pallas_call(
        body,
        grid=(out_rows // bm,),
        in_specs=[pl.BlockSpec((bm, d), blk),
                  pl.BlockSpec((bm, d), blk),
                  pl.BlockSpec((bm, d), blk),
                  pl.BlockSpec((bm, 1), blk),
                  pl.BlockSpec((1, d), lambda i: (0, 0))],
        out_specs=pl.BlockSpec((bm, d), blk),
        out_shape=jax.ShapeDtypeStruct((out_rows, d), jnp.float32),
    )(p0, p1, y, dis, b)


def _tc_split_t(z, bm=640):
    npad, d = z.shape

    def body(z_ref, zh_ref, zl_ref, zht_ref, zlt_ref):
        v = z_ref[...]
        h = v.astype(jnp.bfloat16)
        l = (v - h.astype(jnp.float32)).astype(jnp.bfloat16)
        zh_ref[...] = h
        zl_ref[...] = l
        zht_ref[...] = h.T
        zlt_ref[...] = l.T

    return pl.pallas_call(
        body,
        grid=(npad // bm,),
        in_specs=[pl.BlockSpec((bm, d), lambda i: (i, 0))],
        out_specs=[pl.BlockSpec((bm, d), lambda i: (i, 0)),
                   pl.BlockSpec((bm, d), lambda i: (i, 0)),
                   pl.BlockSpec((d, bm), lambda i: (0, i)),
                   pl.BlockSpec((d, bm), lambda i: (0, i))],
        out_shape=[jax.ShapeDtypeStruct((npad, d), jnp.bfloat16),
                   jax.ShapeDtypeStruct((npad, d), jnp.bfloat16),
                   jax.ShapeDtypeStruct((d, npad), jnp.bfloat16),
                   jax.ShapeDtypeStruct((d, npad), jnp.bfloat16)],
    )(z)


def _tc_ahat(zh, zl, zht, zlt, n, bm=200):
    d = zh.shape[1]

    def body(zh_ref, zl_ref, zht_ref, zlt_ref, o_ref):
        dims = (((1,), (0,)), ((), ()))
        hh = lax.dot_general(zh_ref[...], zht_ref[...], dims,
                             preferred_element_type=jnp.float32)
        hl = lax.dot_general(zh_ref[...], zlt_ref[...], dims,
                             preferred_element_type=jnp.float32)
        lh = lax.dot_general(zl_ref[...], zht_ref[...], dims,
                             preferred_element_type=jnp.float32)
        o_ref[...] = hh + hl + lh

    return pl.pallas_call(
        body,
        grid=(n // bm,),
        in_specs=[pl.BlockSpec((bm, d), lambda i: (i, 0)),
                  pl.BlockSpec((bm, d), lambda i: (i, 0)),
                  pl.BlockSpec((d, n), lambda i: (0, 0)),
                  pl.BlockSpec((d, n), lambda i: (0, 0))],
        out_specs=pl.BlockSpec((bm, n), lambda i: (i, 0)),
        out_shape=jax.ShapeDtypeStruct((n, n), jnp.float32),
    )(zh, zl, zht, zlt)


def kernel(x, edge_index, W1, b1, W2, b2, W3, b3):
    n, d_in = x.shape
    e = edge_index.shape[1]
    d_h = W1.shape[1]

    npad = ((n + NW * 16 - 1) // (NW * 16)) * (NW * 16)
    unit = NW * CHUNK * 8
    epad = ((e + unit - 1) // unit) * unit
    k_per_w = epad // (NW * CHUNK)

    src = edge_index[0].astype(jnp.int32)
    dst = edge_index[1].astype(jnp.int32)
    pad_src = jnp.full((epad - e,), n, jnp.int32)
    pad_dst = n + (jnp.arange(epad - e, dtype=jnp.int32) % (npad - n))
    src_flat = jnp.concatenate([src, pad_src])
    dst_flat = jnp.concatenate([dst, pad_dst])
    src2 = src_flat.reshape(-1, CHUNK)
    dst2 = dst_flat.reshape(-1, CHUNK)
    CH3 = 64
    src3 = src_flat.reshape(-1, CH3)
    dst3 = dst_flat.reshape(-1, CH3)
    k3_per_w = epad // (NW * CH3)

    x_pad = jnp.pad(x, ((0, npad - n), (0, 0)))
    ones_t = jnp.ones((npad, DEG_W), jnp.float32)
    zeros_d = jnp.zeros((npad, DEG_W), jnp.float32)
    zeros_h = jnp.zeros((npad, d_h), jnp.float32)
    zeros_in = jnp.zeros((npad, d_in), jnp.float32)

    g0, g1 = _sc_edge_aggregate(ones_t, dst2, dst2, zeros_d, k_per_w, CHUNK, 4)
    dis = _tc_dis(g0, g1)

    y1 = _tc_mm_scale(x_pad, W1, dis)
    p0, p1 = _sc_edge_aggregate(y1, src2, dst2, zeros_h, k_per_w, CHUNK, 4)
    h1 = _tc_fuse(p0, p1, y1, dis, b1.reshape(1, -1), npad, 1024)

    y2 = _tc_mm_scale(h1, W2, dis)
    q0, q1 = _sc_edge_aggregate(y2, src2, dst2, zeros_h, k_per_w, CHUNK, 4)
    z = _tc_fuse(q0, q1, y2, dis, b2.reshape(1, -1), npad, 1024)

    y3 = _tc_mm_scale(z, W3, dis)
    r0, r1 = _sc_edge_aggregate(y3, src3, dst3, zeros_in, k3_per_w, CH3, 3)
    x_hat = _tc_fuse(r0, r1, y3, dis, b3.reshape(1, -1), n, 400)

    zh, zl, zht, zlt = _tc_split_t(z)
    a_hat = _tc_ahat(zh, zl, zht[:, :n], zlt[:, :n], n)
    return (a_hat, x_hat)


# speedup vs baseline: 22.5168x; 2.4151x over previous
import functools

import jax
import jax.numpy as jnp
from jax import lax
from jax.experimental import pallas as pl
from jax.experimental.pallas import tpu as pltpu
from jax.experimental.pallas import tpu_sc as plsc

NC = 2
NS = 16
NW = NC * NS
CHUNK = 128


def _sc_mesh():
    return plsc.VectorSubcoreMesh(
        core_axis_name="c", subcore_axis_name="s",
        num_cores=NC, num_subcores=NS)


def _sc_edge_aggregate(y, src2, dst2, zeros, k_per_w, chunk, nb):
    npad, d = y.shape
    rows_per_sub = npad // NS
    out_t = jax.ShapeDtypeStruct((npad, d), jnp.float32)

    @functools.partial(
        pl.kernel,
        mesh=_sc_mesh(),
        out_type=(out_t, out_t),
        compiler_params=pltpu.CompilerParams(use_tc_tiling_on_sc=False),
        scratch_types=(
            [
                pltpu.VMEM_SHARED((npad, d), jnp.float32),
                pltpu.VMEM((k_per_w, chunk), jnp.int32),
                pltpu.VMEM((k_per_w, chunk), jnp.int32),
            ]
            + [pltpu.VMEM((chunk, d), jnp.float32)] * nb
            + [pltpu.SemaphoreType.DMA] * nb
        ),
    )
    def body(y_hbm, s_hbm, d_hbm, z_hbm, out0, out1, acc, sidx, didx, *ring):
        bufs = ring[:nb]
        sems = ring[nb:]
        cid = lax.axis_index("c")
        sid = lax.axis_index("s")
        w = cid * NS + sid
        rows = pl.ds(sid * rows_per_sub, rows_per_sub)
        pltpu.sync_copy(z_hbm.at[rows], acc.at[rows])
        pltpu.sync_copy(s_hbm.at[pl.ds(w * k_per_w, k_per_w)], sidx)
        pltpu.sync_copy(d_hbm.at[pl.ds(w * k_per_w, k_per_w)], didx)
        plsc.subcore_barrier()

        for j in range(min(nb, k_per_w)):
            pltpu.async_copy(y_hbm.at[sidx.at[j]], bufs[j % nb], sems[j % nb])
        for j in range(k_per_w):
            b = j % nb
            pltpu.make_async_copy(y_hbm.at[sidx.at[j]], bufs[b],
                                  sems[b]).wait()
            pltpu.sync_copy(bufs[b], acc.at[didx.at[j]], add=True)
            if j + nb < k_per_w:
                pltpu.async_copy(y_hbm.at[sidx.at[j + nb]], bufs[b], sems[b])
        plsc.subcore_barrier()

        @pl.when(cid == 0)
        def _():
            pltpu.sync_copy(acc.at[rows], out0.at[rows])

        @pl.when(cid == 1)
        def _():
            pltpu.sync_copy(acc.at[rows], out1.at[rows])

    return body(y, src2, dst2, zeros)


DEG_W = 8


def _tc_dis(d0, d1, bm=640):
    npad = d0.shape[0]

    def body(a_ref, b_ref, o_ref):
        o_ref[...] = lax.rsqrt(a_ref[...][:, :1] + b_ref[...][:, :1] + 1.0)

    return pl.pallas_call(
        body,
        grid=(npad // bm,),
        in_specs=[pl.BlockSpec((bm, DEG_W), lambda i: (i, 0)),
                  pl.BlockSpec((bm, DEG_W), lambda i: (i, 0))],
        out_specs=pl.BlockSpec((bm, 1), lambda i: (i, 0)),
        out_shape=jax.ShapeDtypeStruct((npad, 1), jnp.float32),
    )(d0, d1)


def _tc_mm_scale(h, w, dis, bm=1024):
    npad, din = h.shape
    dout = w.shape[1]

    def body(h_ref, w_ref, dis_ref, o_ref):
        acc = lax.dot_general(
            h_ref[...], w_ref[...], (((1,), (0,)), ((), ())),
            preferred_element_type=jnp.float32,
            precision=lax.Precision.HIGHEST)
        o_ref[...] = acc * dis_ref[...]

    return pl.pallas_call(
        body,
        grid=(npad // bm,),
        in_specs=[pl.BlockSpec((bm, din), lambda i: (i, 0)),
                  pl.BlockSpec((din, dout), lambda i: (0, 0)),
                  pl.BlockSpec((bm, 1), lambda i: (i, 0))],
        out_specs=pl.BlockSpec((bm, dout), lambda i: (i, 0)),
        out_shape=jax.ShapeDtypeStruct((npad, dout), jnp.float32),
    )(h, w, dis)


def _tc_fuse(p0, p1, y, dis, b, out_rows, bm):
    d = y.shape[1]

    def body(a_ref, b_ref, y_ref, dis_ref, bias_ref, o_ref):
        v = dis_ref[...] * (a_ref[...] + b_ref[...] + y_ref[...])
        o_ref[...] = jnp.maximum(v + bias_ref[...], 0.0)

    blk = lambda i: (i, 0)
    return pl.pallas_call(
        body,
        grid=(out_rows // bm,),
        in_specs=[pl.BlockSpec((bm, d), blk),
                  pl.BlockSpec((bm, d), blk),
                  pl.BlockSpec((bm, d), blk),
                  pl.BlockSpec((bm, 1), blk),
                  pl.BlockSpec((1, d), lambda i: (0, 0))],
        out_specs=pl.BlockSpec((bm, d), blk),
        out_shape=jax.ShapeDtypeStruct((out_rows, d), jnp.float32),
    )(p0, p1, y, dis, b)


def _tc_split_t(z, bm=640):
    npad, d = z.shape

    def body(z_ref, zh_ref, zl_ref, zht_ref, zlt_ref):
        v = z_ref[...]
        h = v.astype(jnp.bfloat16)
        l = (v - h.astype(jnp.float32)).astype(jnp.bfloat16)
        zh_ref[...] = h
        zl_ref[...] = l
        zht_ref[...] = h.T
        zlt_ref[...] = l.T

    return pl.pallas_call(
        body,
        grid=(npad // bm,),
        in_specs=[pl.BlockSpec((bm, d), lambda i: (i, 0))],
        out_specs=[pl.BlockSpec((bm, d), lambda i: (i, 0)),
                   pl.BlockSpec((bm, d), lambda i: (i, 0)),
                   pl.BlockSpec((d, bm), lambda i: (0, i)),
                   pl.BlockSpec((d, bm), lambda i: (0, i))],
        out_shape=[jax.ShapeDtypeStruct((npad, d), jnp.bfloat16),
                   jax.ShapeDtypeStruct((npad, d), jnp.bfloat16),
                   jax.ShapeDtypeStruct((d, npad), jnp.bfloat16),
                   jax.ShapeDtypeStruct((d, npad), jnp.bfloat16)],
    )(z)


def _tc_ahat(zh, zl, zht, zlt, n, bm=200):
    d = zh.shape[1]

    def body(zh_ref, zl_ref, zht_ref, zlt_ref, o_ref):
        dims = (((1,), (0,)), ((), ()))
        hh = lax.dot_general(zh_ref[...], zht_ref[...], dims,
                             preferred_element_type=jnp.float32)
        hl = lax.dot_general(zh_ref[...], zlt_ref[...], dims,
                             preferred_element_type=jnp.float32)
        lh = lax.dot_general(zl_ref[...], zht_ref[...], dims,
                             preferred_element_type=jnp.float32)
        o_ref[...] = hh + hl + lh

    return pl.pallas_call(
        body,
        grid=(n // bm,),
        in_specs=[pl.BlockSpec((bm, d), lambda i: (i, 0)),
                  pl.BlockSpec((bm, d), lambda i: (i, 0)),
                  pl.BlockSpec((d, n), lambda i: (0, 0)),
                  pl.BlockSpec((d, n), lambda i: (0, 0))],
        out_specs=pl.BlockSpec((bm, n), lambda i: (i, 0)),
        out_shape=jax.ShapeDtypeStruct((n, n), jnp.float32),
    )(zh, zl, zht, zlt)


def kernel(x, edge_index, W1, b1, W2, b2, W3, b3):
    n, d_in = x.shape
    e = edge_index.shape[1]
    d_h = W1.shape[1]

    npad = ((n + NW * 16 - 1) // (NW * 16)) * (NW * 16)
    unit = NW * CHUNK * 8
    epad = ((e + unit - 1) // unit) * unit
    k_per_w = epad // (NW * CHUNK)

    src = edge_index[0].astype(jnp.int32)
    dst = edge_index[1].astype(jnp.int32)
    pad_idx = n + (jnp.arange(epad - e, dtype=jnp.int32) % (npad - n))
    pad_src = pad_idx
    pad_dst = pad_idx
    src_flat = jnp.concatenate([src, pad_src])
    dst_flat = jnp.concatenate([dst, pad_dst])
    src2 = src_flat.reshape(-1, CHUNK)
    dst2 = dst_flat.reshape(-1, CHUNK)
    CH3 = 64
    src3 = src_flat.reshape(-1, CH3)
    dst3 = dst_flat.reshape(-1, CH3)
    k3_per_w = epad // (NW * CH3)

    x_pad = jnp.pad(x, ((0, npad - n), (0, 0)))
    ones_t = jnp.ones((npad, DEG_W), jnp.float32)
    zeros_d = jnp.zeros((npad, DEG_W), jnp.float32)
    zeros_h = jnp.zeros((npad, d_h), jnp.float32)
    zeros_in = jnp.zeros((npad, d_in), jnp.float32)

    g0, g1 = _sc_edge_aggregate(ones_t, dst2, dst2, zeros_d, k_per_w, CHUNK, 4)
    dis = _tc_dis(g0, g1)

    y1 = _tc_mm_scale(x_pad, W1, dis)
    p0, p1 = _sc_edge_aggregate(y1, src2, dst2, zeros_h, k_per_w, CHUNK, 4)
    h1 = _tc_fuse(p0, p1, y1, dis, b1.reshape(1, -1), npad, 1024)

    y2 = _tc_mm_scale(h1, W2, dis)
    q0, q1 = _sc_edge_aggregate(y2, src2, dst2, zeros_h, k_per_w, CHUNK, 4)
    z = _tc_fuse(q0, q1, y2, dis, b2.reshape(1, -1), npad, 1024)

    y3 = _tc_mm_scale(z, W3, dis)
    r0, r1 = _sc_edge_aggregate(y3, src3, dst3, zeros_in, k3_per_w, CH3, 3)
    x_hat = _tc_fuse(r0, r1, y3, dis, b3.reshape(1, -1), n, 400)

    zh, zl, zht, zlt = _tc_split_t(z)
    a_hat = _tc_ahat(zh, zl, zht[:, :n], zlt[:, :n], n)
    return (a_hat, x_hat)


# speedup vs baseline: 23.7836x; 1.0563x over previous
import functools

import jax
import jax.numpy as jnp
from jax import lax
from jax.experimental import pallas as pl
from jax.experimental.pallas import tpu as pltpu
from jax.experimental.pallas import tpu_sc as plsc

NC = 2
NS = 16
NW = NC * NS
CHUNK = 128


def _sc_mesh():
    return plsc.VectorSubcoreMesh(
        core_axis_name="c", subcore_axis_name="s",
        num_cores=NC, num_subcores=NS)


def _sc_edge_aggregate(y, src2, dst2, zeros, k_per_w, chunk, nb):
    npad, d = y.shape
    rows_per_sub = npad // NS
    out_t = jax.ShapeDtypeStruct((npad, d), jnp.float32)

    @functools.partial(
        pl.kernel,
        mesh=_sc_mesh(),
        out_type=(out_t, out_t),
        compiler_params=pltpu.CompilerParams(use_tc_tiling_on_sc=False),
        scratch_types=(
            [
                pltpu.VMEM_SHARED((npad, d), jnp.float32),
                pltpu.VMEM((k_per_w, chunk), jnp.int32),
                pltpu.VMEM((k_per_w, chunk), jnp.int32),
            ]
            + [pltpu.VMEM((chunk, d), jnp.float32)] * nb
            + [pltpu.SemaphoreType.DMA] * nb
        ),
    )
    def body(y_hbm, s_hbm, d_hbm, z_hbm, out0, out1, acc, sidx, didx, *ring):
        bufs = ring[:nb]
        sems = ring[nb:]
        cid = lax.axis_index("c")
        sid = lax.axis_index("s")
        w = cid * NS + sid
        rows = pl.ds(sid * rows_per_sub, rows_per_sub)
        pltpu.sync_copy(z_hbm.at[rows], acc.at[rows])
        pltpu.sync_copy(s_hbm.at[pl.ds(w * k_per_w, k_per_w)], sidx)
        pltpu.sync_copy(d_hbm.at[pl.ds(w * k_per_w, k_per_w)], didx)
        plsc.subcore_barrier()

        for j in range(min(nb, k_per_w)):
            pltpu.async_copy(y_hbm.at[sidx.at[j]], bufs[j % nb], sems[j % nb])
        for j in range(k_per_w):
            b = j % nb
            pltpu.make_async_copy(y_hbm.at[sidx.at[j]], bufs[b],
                                  sems[b]).wait()
            pltpu.sync_copy(bufs[b], acc.at[didx.at[j]], add=True)
            if j + nb < k_per_w:
                pltpu.async_copy(y_hbm.at[sidx.at[j + nb]], bufs[b], sems[b])
        plsc.subcore_barrier()

        @pl.when(cid == 0)
        def _():
            pltpu.sync_copy(acc.at[rows], out0.at[rows])

        @pl.when(cid == 1)
        def _():
            pltpu.sync_copy(acc.at[rows], out1.at[rows])

    return body(y, src2, dst2, zeros)


DEG_W = 8


def _tc_dis_mm(d0, d1, x, w, bm=1024):
    npad, din = x.shape
    dout = w.shape[1]

    def body(a_ref, b_ref, x_ref, w_ref, dis_ref, y_ref):
        dis = lax.rsqrt(a_ref[...][:, :1] + b_ref[...][:, :1] + 1.0)
        dis_ref[...] = dis
        acc = lax.dot_general(
            x_ref[...], w_ref[...], (((1,), (0,)), ((), ())),
            preferred_element_type=jnp.float32,
            precision=lax.Precision.HIGHEST)
        y_ref[...] = acc * dis

    return pl.pallas_call(
        body,
        grid=(npad // bm,),
        in_specs=[pl.BlockSpec((bm, DEG_W), lambda i: (i, 0)),
                  pl.BlockSpec((bm, DEG_W), lambda i: (i, 0)),
                  pl.BlockSpec((bm, din), lambda i: (i, 0)),
                  pl.BlockSpec((din, dout), lambda i: (0, 0))],
        out_specs=[pl.BlockSpec((bm, 1), lambda i: (i, 0)),
                   pl.BlockSpec((bm, dout), lambda i: (i, 0))],
        out_shape=[jax.ShapeDtypeStruct((npad, 1), jnp.float32),
                   jax.ShapeDtypeStruct((npad, dout), jnp.float32)],
    )(d0, d1, x, w)


def _tc_fuse_mm(p0, p1, y, dis, b, w, bm=1024):
    d = y.shape[1]
    npad = y.shape[0]
    dout = w.shape[1]

    def body(a_ref, b_ref, y_ref, dis_ref, bias_ref, w_ref, h_ref, o_ref):
        dis = dis_ref[...]
        v = dis * (a_ref[...] + b_ref[...] + y_ref[...])
        h = jnp.maximum(v + bias_ref[...], 0.0)
        h_ref[...] = h
        acc = lax.dot_general(
            h, w_ref[...], (((1,), (0,)), ((), ())),
            preferred_element_type=jnp.float32,
            precision=lax.Precision.HIGHEST)
        o_ref[...] = acc * dis

    blk = lambda i: (i, 0)
    return pl.pallas_call(
        body,
        grid=(npad // bm,),
        in_specs=[pl.BlockSpec((bm, d), blk),
                  pl.BlockSpec((bm, d), blk),
                  pl.BlockSpec((bm, d), blk),
                  pl.BlockSpec((bm, 1), blk),
                  pl.BlockSpec((1, d), lambda i: (0, 0)),
                  pl.BlockSpec((d, dout), lambda i: (0, 0))],
        out_specs=[pl.BlockSpec((bm, d), blk),
                   pl.BlockSpec((bm, dout), blk)],
        out_shape=[jax.ShapeDtypeStruct((npad, d), jnp.float32),
                   jax.ShapeDtypeStruct((npad, dout), jnp.float32)],
    )(p0, p1, y, dis, b, w)


def _tc_fuse(p0, p1, y, dis, b, out_rows, bm):
    d = y.shape[1]

    def body(a_ref, b_ref, y_ref, dis_ref, bias_ref, o_ref):
        v = dis_ref[...] * (a_ref[...] + b_ref[...] + y_ref[...])
        o_ref[...] = jnp.maximum(v + bias_ref[...], 0.0)

    blk = lambda i: (i, 0)
    return pl.pallas_call(
        body,
        grid=(out_rows // bm,),
        in_specs=[pl.BlockSpec((bm, d), blk),
                  pl.BlockSpec((bm, d), blk),
                  pl.BlockSpec((bm, d), blk),
                  pl.BlockSpec((bm, 1), blk),
                  pl.BlockSpec((1, d), lambda i: (0, 0))],
        out_specs=pl.BlockSpec((bm, d), blk),
        out_shape=jax.ShapeDtypeStruct((out_rows, d), jnp.float32),
    )(p0, p1, y, dis, b)


def _tc_split_t(z, bm=640):
    npad, d = z.shape

    def body(z_ref, zh_ref, zl_ref, zht_ref, zlt_ref):
        v = z_ref[...]
        h = v.astype(jnp.bfloat16)
        l = (v - h.astype(jnp.float32)).astype(jnp.bfloat16)
        zh_ref[...] = h
        zl_ref[...] = l
        zht_ref[...] = h.T
        zlt_ref[...] = l.T

    return pl.pallas_call(
        body,
        grid=(npad // bm,),
        in_specs=[pl.BlockSpec((bm, d), lambda i: (i, 0))],
        out_specs=[pl.BlockSpec((bm, d), lambda i: (i, 0)),
                   pl.BlockSpec((bm, d), lambda i: (i, 0)),
                   pl.BlockSpec((d, bm), lambda i: (0, i)),
                   pl.BlockSpec((d, bm), lambda i: (0, i))],
        out_shape=[jax.ShapeDtypeStruct((npad, d), jnp.bfloat16),
                   jax.ShapeDtypeStruct((npad, d), jnp.bfloat16),
                   jax.ShapeDtypeStruct((d, npad), jnp.bfloat16),
                   jax.ShapeDtypeStruct((d, npad), jnp.bfloat16)],
    )(z)


def _tc_ahat(zh, zl, zht, zlt, n, bm=200):
    d = zh.shape[1]

    def body(zh_ref, zl_ref, zht_ref, zlt_ref, o_ref):
        dims = (((1,), (0,)), ((), ()))
        hh = lax.dot_general(zh_ref[...], zht_ref[...], dims,
                             preferred_element_type=jnp.float32)
        hl = lax.dot_general(zh_ref[...], zlt_ref[...], dims,
                             preferred_element_type=jnp.float32)
        lh = lax.dot_general(zl_ref[...], zht_ref[...], dims,
                             preferred_element_type=jnp.float32)
        o_ref[...] = hh + hl + lh

    return pl.pallas_call(
        body,
        grid=(n // bm,),
        in_specs=[pl.BlockSpec((bm, d), lambda i: (i, 0)),
                  pl.BlockSpec((bm, d), lambda i: (i, 0)),
                  pl.BlockSpec((d, n), lambda i: (0, 0)),
                  pl.BlockSpec((d, n), lambda i: (0, 0))],
        out_specs=pl.BlockSpec((bm, n), lambda i: (i, 0)),
        out_shape=jax.ShapeDtypeStruct((n, n), jnp.float32),
    )(zh, zl, zht, zlt)


def kernel(x, edge_index, W1, b1, W2, b2, W3, b3):
    n, d_in = x.shape
    e = edge_index.shape[1]
    d_h = W1.shape[1]

    npad = ((n + NW * 16 - 1) // (NW * 16)) * (NW * 16)
    unit = NW * CHUNK * 8
    epad = ((e + unit - 1) // unit) * unit
    k_per_w = epad // (NW * CHUNK)

    src = edge_index[0].astype(jnp.int32)
    dst = edge_index[1].astype(jnp.int32)
    pad_idx = n + (jnp.arange(epad - e, dtype=jnp.int32) % (npad - n))
    pad_src = pad_idx
    pad_dst = pad_idx
    src_flat = jnp.concatenate([src, pad_src])
    dst_flat = jnp.concatenate([dst, pad_dst])
    src2 = src_flat.reshape(-1, CHUNK)
    dst2 = dst_flat.reshape(-1, CHUNK)
    CH3 = 64
    src3 = src_flat.reshape(-1, CH3)
    dst3 = dst_flat.reshape(-1, CH3)
    k3_per_w = epad // (NW * CH3)

    x_pad = jnp.pad(x, ((0, npad - n), (0, 0)))
    ones_t = jnp.ones((npad, DEG_W), jnp.float32)
    zeros_d = jnp.zeros((npad, DEG_W), jnp.float32)
    zeros_h = jnp.zeros((npad, d_h), jnp.float32)
    zeros_in = jnp.zeros((npad, d_in), jnp.float32)

    g0, g1 = _sc_edge_aggregate(ones_t, dst2, dst2, zeros_d, k_per_w, CHUNK, 4)
    dis, y1 = _tc_dis_mm(g0, g1, x_pad, W1)

    p0, p1 = _sc_edge_aggregate(y1, src2, dst2, zeros_h, k_per_w, CHUNK, 4)
    _, y2 = _tc_fuse_mm(p0, p1, y1, dis, b1.reshape(1, -1), W2)

    q0, q1 = _sc_edge_aggregate(y2, src2, dst2, zeros_h, k_per_w, CHUNK, 4)
    z, y3 = _tc_fuse_mm(q0, q1, y2, dis, b2.reshape(1, -1), W3)

    r0, r1 = _sc_edge_aggregate(y3, src3, dst3, zeros_in, k3_per_w, CH3, 3)
    x_hat = _tc_fuse(r0, r1, y3, dis, b3.reshape(1, -1), n, 400)

    zh, zl, zht, zlt = _tc_split_t(z)
    a_hat = _tc_ahat(zh, zl, zht[:, :n], zlt[:, :n], n)
    return (a_hat, x_hat)
